# baseline (speedup 1.0000x reference)
import functools

import jax
import jax.numpy as jnp
from jax import lax
from jax.experimental import pallas as pl
from jax.experimental.pallas import tpu as pltpu
from jax.experimental.pallas import tpu_sc as plsc

N_B = 8
T_LEN = 128
FNUM = 16
W_LEN = 112
N_SLABS = N_B * 3 * 112
N_ROWS = N_SLABS * W_LEN
N_TILES = 32
SLABS_PER_TILE = N_SLABS // N_TILES
CHUNK_SLABS = 3
N_CHUNKS = SLABS_PER_TILE // CHUNK_SLABS
CHUNK_ROWS = CHUNK_SLABS * W_LEN
SLAB_OUT = FNUM * T_LEN


def _sc_gather(xt_flat, gidx):
  mesh = plsc.VectorSubcoreMesh(core_axis_name="c", subcore_axis_name="s")

  @functools.partial(
      pl.kernel,
      mesh=mesh,
      out_type=jax.ShapeDtypeStruct((N_SLABS * SLAB_OUT,), jnp.float32),
      scratch_types=[
          pltpu.VMEM((SLABS_PER_TILE * FNUM,), jnp.int32),
          pltpu.VMEM((CHUNK_ROWS * T_LEN,), jnp.float32),
          pltpu.VMEM((CHUNK_ROWS * T_LEN,), jnp.float32),
          pltpu.VMEM((CHUNK_SLABS * SLAB_OUT,), jnp.float32),
          pltpu.VMEM((CHUNK_SLABS * SLAB_OUT,), jnp.float32),
          pltpu.SemaphoreType.DMA,
          pltpu.SemaphoreType.DMA,
          pltpu.SemaphoreType.DMA,
          pltpu.SemaphoreType.DMA,
      ],
      compiler_params=pltpu.CompilerParams(needs_layout_passes=False),
  )
  def k(xt_hbm, gidx_hbm, out_hbm, idx_v, in_a, in_b, out_a, out_b,
        gs_a, gs_b, ss_a, ss_b):
    wid = lax.axis_index("s") * 2 + lax.axis_index("c")
    slab0 = wid * SLABS_PER_TILE
    pltpu.sync_copy(
        gidx_hbm.at[pl.ds(slab0 * FNUM, SLABS_PER_TILE * FNUM)], idx_v)
    lane = lax.iota(jnp.int32, FNUM)
    ins = (in_a, in_b)
    outs = (out_a, out_b)
    gsems = (gs_a, gs_b)
    ssems = (ss_a, ss_b)

    gathers = [None] * N_CHUNKS
    scatters = [None, None]

    def start_gather(c):
      slot = c % 2
      return pltpu.async_copy(
          xt_hbm.at[pl.ds((slab0 + c * CHUNK_SLABS) * W_LEN * T_LEN,
                          CHUNK_ROWS * T_LEN)],
          ins[slot], gsems[slot])

    gathers[0] = start_gather(0)
    gathers[1] = start_gather(1)
    for c in range(N_CHUNKS):
      slot = c % 2
      in_buf, out_buf = ins[slot], outs[slot]
      gathers[c].wait()
      if scatters[slot] is not None:
        scatters[slot].wait()
      for s in range(CHUNK_SLABS):
        tvec = idx_v[pl.ds((c * CHUNK_SLABS + s) * FNUM, FNUM)]
        gaddr0 = tvec + jnp.int32(s * W_LEN * T_LEN)
        sidx0 = lane * T_LEN + jnp.int32(s * SLAB_OUT)

        @plsc.parallel_loop(0, W_LEN, 1, unroll=8,
                            carry=(gaddr0, sidx0))
        def body(w, carry):
          gaddr, sidx = carry
          vals = plsc.load_gather(in_buf, [gaddr])
          plsc.store_scatter(out_buf, [sidx], vals)
          return (gaddr + T_LEN, sidx + 1)

      scatters[slot] = pltpu.async_copy(
          out_buf,
          out_hbm.at[pl.ds((slab0 + c * CHUNK_SLABS) * SLAB_OUT,
                           CHUNK_SLABS * SLAB_OUT)],
          ssems[slot])
      if c + 2 < N_CHUNKS:
        gathers[c + 2] = start_gather(c + 2)
    scatters[0].wait()
    scatters[1].wait()

  return k(xt_flat, gidx)


def _tc_relayout(y8):
  def body(in_ref, out_ref):
    blk = in_ref[0, 0]
    out_ref[0, 0, 0] = blk.reshape(112, T_LEN)[:, :W_LEN]

  return pl.pallas_call(
      body,
      grid=(N_B, FNUM, 3),
      in_specs=[pl.BlockSpec(
          (1, 1, 14, 8, T_LEN),
          lambda b, f, c: (b, c, 0, 0, f))],
      out_specs=pl.BlockSpec(
          (1, 1, 1, 112, W_LEN),
          lambda b, f, c: (b, f, c, 0, 0)),
      out_shape=jax.ShapeDtypeStruct((N_B, FNUM, 3, 112, 112), jnp.float32),
  )(y8)


def kernel(x):
  B, T = x.shape[0], x.shape[1]
  fnum = FNUM
  start, end = 0, T - 1
  fid_base = jnp.linspace(start, end, fnum).astype(jnp.int32)
  step = int((end - start) / fnum)
  if step != 0:
    key = jax.random.key(42)
    y = jax.random.randint(key, (B, fnum), 0, step, dtype=jnp.int32)
    y = y.at[:, fnum - 1].set(0)
  else:
    y = jnp.zeros((B, fnum), dtype=jnp.int32)
  fid = fid_base[None, :] + y
  slab_b = jnp.arange(N_SLABS, dtype=jnp.int32) // (N_SLABS // N_B)
  gidx = fid[slab_b].reshape(-1)
  xt_flat = jnp.transpose(x, (0, 2, 3, 4, 1)).reshape(N_ROWS * T_LEN)
  out_flat = _sc_gather(xt_flat, gidx)
  y8 = out_flat.reshape(N_B, 3, 14, 8, FNUM * T_LEN)
  return _tc_relayout(y8)


# speedup vs baseline: 1.0649x; 1.0649x over previous
import functools

import jax
import jax.numpy as jnp
from jax import lax
from jax.experimental import pallas as pl
from jax.experimental.pallas import tpu as pltpu
from jax.experimental.pallas import tpu_sc as plsc

N_B = 8
T_LEN = 128
FNUM = 16
W_LEN = 112
N_SLABS = N_B * 3 * 112
N_ROWS = N_SLABS * W_LEN
N_TILES = 32
SLABS_PER_TILE = N_SLABS // N_TILES
CHUNK_SLABS = 3
N_CHUNKS = SLABS_PER_TILE // CHUNK_SLABS
CHUNK_ROWS = CHUNK_SLABS * W_LEN
SLAB_OUT = FNUM * T_LEN


def _sc_gather(xt_flat, gidx):
  mesh = plsc.VectorSubcoreMesh(core_axis_name="c", subcore_axis_name="s")

  @functools.partial(
      pl.kernel,
      mesh=mesh,
      out_type=jax.ShapeDtypeStruct((N_SLABS * SLAB_OUT,), jnp.float32),
      scratch_types=[
          pltpu.VMEM((SLABS_PER_TILE * FNUM,), jnp.int32),
          pltpu.VMEM((CHUNK_ROWS * T_LEN,), jnp.float32),
          pltpu.VMEM((CHUNK_ROWS * T_LEN,), jnp.float32),
          pltpu.VMEM((CHUNK_SLABS * SLAB_OUT,), jnp.float32),
          pltpu.VMEM((CHUNK_SLABS * SLAB_OUT,), jnp.float32),
          pltpu.SemaphoreType.DMA,
          pltpu.SemaphoreType.DMA,
          pltpu.SemaphoreType.DMA,
          pltpu.SemaphoreType.DMA,
      ],
      compiler_params=pltpu.CompilerParams(needs_layout_passes=False),
  )
  def k(xt_hbm, gidx_hbm, out_hbm, idx_v, in_a, in_b, out_a, out_b,
        gs_a, gs_b, ss_a, ss_b):
    wid = lax.axis_index("s") * 2 + lax.axis_index("c")
    slab0 = wid * SLABS_PER_TILE
    pltpu.sync_copy(
        gidx_hbm.at[pl.ds(slab0 * FNUM, SLABS_PER_TILE * FNUM)], idx_v)
    lane = lax.iota(jnp.int32, FNUM)
    ins = (in_a, in_b)
    outs = (out_a, out_b)
    gsems = (gs_a, gs_b)
    ssems = (ss_a, ss_b)

    gathers = [None] * N_CHUNKS
    scatters = [None, None]

    def start_gather(c):
      slot = c % 2
      return pltpu.async_copy(
          xt_hbm.at[pl.ds((slab0 + c * CHUNK_SLABS) * W_LEN * T_LEN,
                          CHUNK_ROWS * T_LEN)],
          ins[slot], gsems[slot])

    gathers[0] = start_gather(0)
    gathers[1] = start_gather(1)
    for c in range(N_CHUNKS):
      slot = c % 2
      in_buf, out_buf = ins[slot], outs[slot]
      gathers[c].wait()
      if scatters[slot] is not None:
        scatters[slot].wait()
      for s in range(CHUNK_SLABS):
        tvec = idx_v[pl.ds((c * CHUNK_SLABS + s) * FNUM, FNUM)]
        gaddr0 = tvec + jnp.int32(s * W_LEN * T_LEN)
        sidx0 = lane * T_LEN + jnp.int32(s * SLAB_OUT)

        @plsc.parallel_loop(0, W_LEN, 1, unroll=8,
                            carry=(gaddr0, sidx0))
        def body(w, carry):
          gaddr, sidx = carry
          vals = plsc.load_gather(in_buf, [gaddr])
          plsc.store_scatter(out_buf, [sidx], vals)
          return (gaddr + T_LEN, sidx + 1)

      scatters[slot] = pltpu.async_copy(
          out_buf,
          out_hbm.at[pl.ds((slab0 + c * CHUNK_SLABS) * SLAB_OUT,
                           CHUNK_SLABS * SLAB_OUT)],
          ssems[slot])
      if c + 2 < N_CHUNKS:
        gathers[c + 2] = start_gather(c + 2)
    scatters[0].wait()
    scatters[1].wait()

  return k(xt_flat, gidx)


def _tc_relayout(y8):
  def body(in_ref, out_ref, p_ref):
    b = pl.program_id(0)
    c = pl.program_id(1)
    hb = pl.program_id(2)

    @pl.when(jnp.logical_and(b == 0, jnp.logical_and(c == 0, hb == 0)))
    def _():
      r = lax.broadcasted_iota(jnp.int32, (128, 128), 0)
      col = lax.broadcasted_iota(jnp.int32, (128, 128), 1)
      p_ref[...] = jnp.where((r % 8) * FNUM + r // 8 == col, 1.0, 0.0)

    a = in_ref[0, 0, 0].reshape(128, T_LEN)
    perm = lax.dot_general(p_ref[...], a, (((1,), (0,)), ((), ())),
                           precision=lax.Precision.HIGHEST,
                           preferred_element_type=jnp.float32)
    out_ref[0, :, 0] = perm.reshape(FNUM, 8, T_LEN)[:, :, :W_LEN]

  return pl.pallas_call(
      body,
      grid=(N_B, 3, 14),
      in_specs=[pl.BlockSpec(
          (1, 1, 1, 8, FNUM, T_LEN),
          lambda b, c, hb: (b, c, hb, 0, 0, 0))],
      out_specs=pl.BlockSpec(
          (1, FNUM, 1, 8, W_LEN),
          lambda b, c, hb: (b, 0, c, hb, 0)),
      out_shape=jax.ShapeDtypeStruct((N_B, FNUM, 3, 112, 112), jnp.float32),
      scratch_shapes=[pltpu.VMEM((128, 128), jnp.float32)],
  )(y8)


def kernel(x):
  B, T = x.shape[0], x.shape[1]
  fnum = FNUM
  start, end = 0, T - 1
  fid_base = jnp.linspace(start, end, fnum).astype(jnp.int32)
  step = int((end - start) / fnum)
  if step != 0:
    key = jax.random.key(42)
    y = jax.random.randint(key, (B, fnum), 0, step, dtype=jnp.int32)
    y = y.at[:, fnum - 1].set(0)
  else:
    y = jnp.zeros((B, fnum), dtype=jnp.int32)
  fid = fid_base[None, :] + y
  slab_b = jnp.arange(N_SLABS, dtype=jnp.int32) // (N_SLABS // N_B)
  gidx = fid[slab_b].reshape(-1)
  xt_flat = jnp.transpose(x, (0, 2, 3, 4, 1)).reshape(N_ROWS * T_LEN)
  out_flat = _sc_gather(xt_flat, gidx)
  y8 = out_flat.reshape(N_B, 3, 14, 8, FNUM, T_LEN)
  return _tc_relayout(y8)


# speedup vs baseline: 1.6828x; 1.5801x over previous
import functools

import jax
import jax.numpy as jnp
from jax import lax
from jax.experimental import pallas as pl
from jax.experimental.pallas import tpu as pltpu
from jax.experimental.pallas import tpu_sc as plsc

N_B = 8
T_LEN = 128
FNUM = 16
W_LEN = 112
N_SLABS = N_B * 3 * 112
N_ROWS = N_SLABS * W_LEN
N_TILES = 32
SLABS_PER_TILE = N_SLABS // N_TILES
CHUNK_SLABS = 2
N_CHUNKS = SLABS_PER_TILE // CHUNK_SLABS
CHUNK_ROWS = CHUNK_SLABS * W_LEN
CHUNK_OUT = CHUNK_SLABS * T_LEN
Z_PLANE = 3 * W_LEN * T_LEN
Z_SIZE = N_B * FNUM * Z_PLANE


def _sc_gather(xt_flat, gidx):
  mesh = plsc.VectorSubcoreMesh(core_axis_name="c", subcore_axis_name="s")

  @functools.partial(
      pl.kernel,
      mesh=mesh,
      out_type=jax.ShapeDtypeStruct((Z_SIZE,), jnp.float32),
      scratch_types=[
          pltpu.VMEM((SLABS_PER_TILE * FNUM,), jnp.int32),
          pltpu.VMEM((CHUNK_ROWS * T_LEN,), jnp.float32),
          pltpu.VMEM((CHUNK_ROWS * T_LEN,), jnp.float32),
          pltpu.VMEM((FNUM * CHUNK_OUT,), jnp.float32),
          pltpu.VMEM((FNUM * CHUNK_OUT,), jnp.float32),
          pltpu.SemaphoreType.DMA,
          pltpu.SemaphoreType.DMA,
          pltpu.SemaphoreType.DMA,
          pltpu.SemaphoreType.DMA,
      ],
      compiler_params=pltpu.CompilerParams(needs_layout_passes=False),
  )
  def k(xt_hbm, gidx_hbm, out_hbm, idx_v, in_a, in_b, out_a, out_b,
        gs_a, gs_b, ss_a, ss_b):
    wid = lax.axis_index("s") * 2 + lax.axis_index("c")
    slab0 = wid * SLABS_PER_TILE
    pltpu.sync_copy(
        gidx_hbm.at[pl.ds(slab0 * FNUM, SLABS_PER_TILE * FNUM)], idx_v)
    lane = lax.iota(jnp.int32, FNUM)
    ins = (in_a, in_b)
    outs = (out_a, out_b)
    gsems = (gs_a, gs_b)
    ssems = (ss_a, ss_b)

    gathers = [None] * N_CHUNKS
    scatters = [None, None]

    def start_gather(c):
      slot = c % 2
      return pltpu.async_copy(
          xt_hbm.at[pl.ds((slab0 + c * CHUNK_SLABS) * W_LEN * T_LEN,
                          CHUNK_ROWS * T_LEN)],
          ins[slot], gsems[slot])

    gathers[0] = start_gather(0)
    gathers[1] = start_gather(1)
    for c in range(N_CHUNKS):
      slot = c % 2
      in_buf, out_buf = ins[slot], outs[slot]
      gathers[c].wait()
      if scatters[slot] is not None:
        for w8 in scatters[slot]:
          w8.wait()
      for s in range(CHUNK_SLABS):
        tvec = idx_v[pl.ds((c * CHUNK_SLABS + s) * FNUM, FNUM)]
        gaddr0 = tvec + jnp.int32(s * W_LEN * T_LEN)
        sidx0 = lane * CHUNK_OUT + jnp.int32(s * T_LEN)

        @plsc.parallel_loop(0, W_LEN, 1, unroll=8,
                            carry=(gaddr0, sidx0))
        def body(w, carry):
          gaddr, sidx = carry
          vals = plsc.load_gather(in_buf, [gaddr])
          plsc.store_scatter(out_buf, [sidx], vals)
          return (gaddr + T_LEN, sidx + 1)

      slab = slab0 + c * CHUNK_SLABS
      b = slab // 336
      rem = slab - b * 336
      zbase = b * (FNUM * Z_PLANE) + rem * T_LEN
      sc = []
      for f in range(FNUM):
        sc.append(pltpu.async_copy(
            out_buf.at[pl.ds(f * CHUNK_OUT, CHUNK_OUT)],
            out_hbm.at[pl.ds(zbase + f * Z_PLANE, CHUNK_OUT)],
            ssems[slot]))
      scatters[slot] = sc
      if c + 2 < N_CHUNKS:
        gathers[c + 2] = start_gather(c + 2)
    for sl in scatters:
      if sl is not None:
        for w8 in sl:
          w8.wait()

  return k(xt_flat, gidx)


def _tc_trim(z6):
  def body(in_ref, out_ref):
    out_ref[0, 0] = in_ref[0, 0][:, :, :W_LEN]

  return pl.pallas_call(
      body,
      grid=(N_B, FNUM),
      in_specs=[pl.BlockSpec(
          (1, 1, 3, 112, T_LEN), lambda b, f: (b, f, 0, 0, 0))],
      out_specs=pl.BlockSpec(
          (1, 1, 3, 112, W_LEN), lambda b, f: (b, f, 0, 0, 0)),
      out_shape=jax.ShapeDtypeStruct((N_B, FNUM, 3, 112, 112), jnp.float32),
  )(z6)


def kernel(x):
  B, T = x.shape[0], x.shape[1]
  fnum = FNUM
  start, end = 0, T - 1
  fid_base = jnp.linspace(start, end, fnum).astype(jnp.int32)
  step = int((end - start) / fnum)
  if step != 0:
    key = jax.random.key(42)
    y = jax.random.randint(key, (B, fnum), 0, step, dtype=jnp.int32)
    y = y.at[:, fnum - 1].set(0)
  else:
    y = jnp.zeros((B, fnum), dtype=jnp.int32)
  fid = fid_base[None, :] + y
  slab_b = jnp.arange(N_SLABS, dtype=jnp.int32) // (N_SLABS // N_B)
  gidx = fid[slab_b].reshape(-1)
  xt_flat = jnp.transpose(x, (0, 2, 3, 4, 1)).reshape(N_ROWS * T_LEN)
  z = _sc_gather(xt_flat, gidx)
  z6 = z.reshape(N_B, FNUM, 3, 112, T_LEN)
  return _tc_trim(z6)


# speedup vs baseline: 2.3478x; 1.3952x over previous
import functools

import jax
import jax.numpy as jnp
from jax import lax
from jax.experimental import pallas as pl
from jax.experimental.pallas import tpu as pltpu
from jax.experimental.pallas import tpu_sc as plsc

N_B = 8
T_LEN = 128
FNUM = 16
W_LEN = 112
N_SLABS = N_B * 3 * 112
N_ROWS = N_SLABS * W_LEN
N_TILES = 32
SLABS_PER_TILE = N_SLABS // N_TILES
CHUNK_SLABS = 2
N_CHUNKS = SLABS_PER_TILE // CHUNK_SLABS
CHUNK_ROWS = CHUNK_SLABS * W_LEN
CHUNK_OUT = CHUNK_SLABS * T_LEN
Z_PLANE = 3 * W_LEN * T_LEN
Z_SIZE = N_B * FNUM * Z_PLANE


def _sc_gather(xt_flat, gidx):
  mesh = plsc.VectorSubcoreMesh(core_axis_name="c", subcore_axis_name="s")

  @functools.partial(
      pl.kernel,
      mesh=mesh,
      out_type=jax.ShapeDtypeStruct((Z_SIZE,), jnp.float32),
      scratch_types=[
          pltpu.VMEM((SLABS_PER_TILE * FNUM,), jnp.int32),
          pltpu.VMEM((CHUNK_ROWS * T_LEN,), jnp.float32),
          pltpu.VMEM((CHUNK_ROWS * T_LEN,), jnp.float32),
          pltpu.VMEM((FNUM * CHUNK_OUT,), jnp.float32),
          pltpu.VMEM((FNUM * CHUNK_OUT,), jnp.float32),
          pltpu.SemaphoreType.DMA,
          pltpu.SemaphoreType.DMA,
          pltpu.SemaphoreType.DMA,
          pltpu.SemaphoreType.DMA,
      ],
      compiler_params=pltpu.CompilerParams(needs_layout_passes=False),
  )
  def k(xt_hbm, gidx_hbm, out_hbm, idx_v, in_a, in_b, out_a, out_b,
        gs_a, gs_b, ss_a, ss_b):
    wid = lax.axis_index("s") * 2 + lax.axis_index("c")
    slab0 = wid * SLABS_PER_TILE
    pltpu.sync_copy(
        gidx_hbm.at[pl.ds(slab0 * FNUM, SLABS_PER_TILE * FNUM)], idx_v)
    lane = lax.iota(jnp.int32, FNUM)
    ins = (in_a, in_b)
    outs = (out_a, out_b)
    gsems = (gs_a, gs_b)
    ssems = (ss_a, ss_b)

    gathers = [None] * N_CHUNKS
    scatters = [None, None]

    def start_gather(c):
      slot = c % 2
      return pltpu.async_copy(
          xt_hbm.at[pl.ds((slab0 + c * CHUNK_SLABS) * W_LEN * T_LEN,
                          CHUNK_ROWS * T_LEN)],
          ins[slot], gsems[slot])

    gathers[0] = start_gather(0)
    gathers[1] = start_gather(1)
    for c in range(N_CHUNKS):
      slot = c % 2
      in_buf, out_buf = ins[slot], outs[slot]
      gathers[c].wait()
      if scatters[slot] is not None:
        for w8 in scatters[slot]:
          w8.wait()
      for s in range(CHUNK_SLABS):
        tvec = idx_v[pl.ds((c * CHUNK_SLABS + s) * FNUM, FNUM)]
        gaddr0 = tvec + jnp.int32(s * W_LEN * T_LEN)
        sidx0 = lane * CHUNK_OUT + jnp.int32(s * T_LEN)

        @plsc.parallel_loop(0, W_LEN, 1, unroll=8,
                            carry=(gaddr0, sidx0))
        def body(w, carry):
          gaddr, sidx = carry
          vals = plsc.load_gather(in_buf, [gaddr])
          plsc.store_scatter(out_buf, [sidx], vals)
          return (gaddr + T_LEN, sidx + 1)

      slab = slab0 + c * CHUNK_SLABS
      b = slab // 336
      rem = slab - b * 336
      zbase = b * (FNUM * Z_PLANE) + rem * T_LEN
      sc = []
      for f in range(FNUM):
        sc.append(pltpu.async_copy(
            out_buf.at[pl.ds(f * CHUNK_OUT, CHUNK_OUT)],
            out_hbm.at[pl.ds(zbase + f * Z_PLANE, CHUNK_OUT)],
            ssems[slot]))
      scatters[slot] = sc
      if c + 2 < N_CHUNKS:
        gathers[c + 2] = start_gather(c + 2)
    for sl in scatters:
      if sl is not None:
        for w8 in sl:
          w8.wait()

  return k(xt_flat, gidx)


def _tc_trim(z6):
  def body(in_ref, out_ref):
    out_ref[0] = in_ref[0][:, :, :, :W_LEN]

  return pl.pallas_call(
      body,
      grid=(N_B,),
      in_specs=[pl.BlockSpec(
          (1, FNUM, 3, 112, T_LEN), lambda b: (b, 0, 0, 0, 0))],
      out_specs=pl.BlockSpec(
          (1, FNUM, 3, 112, W_LEN), lambda b: (b, 0, 0, 0, 0)),
      out_shape=jax.ShapeDtypeStruct((N_B, FNUM, 3, 112, 112), jnp.float32),
  )(z6)


def kernel(x):
  B, T = x.shape[0], x.shape[1]
  fnum = FNUM
  start, end = 0, T - 1
  fid_base = jnp.linspace(start, end, fnum).astype(jnp.int32)
  step = int((end - start) / fnum)
  if step != 0:
    key = jax.random.key(42)
    y = jax.random.randint(key, (B, fnum), 0, step, dtype=jnp.int32)
    y = y.at[:, fnum - 1].set(0)
  else:
    y = jnp.zeros((B, fnum), dtype=jnp.int32)
  fid = fid_base[None, :] + y
  slab_b = jnp.arange(N_SLABS, dtype=jnp.int32) // (N_SLABS // N_B)
  gidx = fid[slab_b].reshape(-1)
  xt_flat = jnp.transpose(x, (0, 2, 3, 4, 1)).reshape(N_ROWS * T_LEN)
  z = _sc_gather(xt_flat, gidx)
  z6 = z.reshape(N_B, FNUM, 3, 112, T_LEN)
  return _tc_trim(z6)
